# Initial kernel scaffold; baseline (speedup 1.0000x reference)
import jax
import jax.numpy as jnp
from jax.experimental import pallas as pl


def kernel(inputs, embedding_matrix):
    raise NotImplementedError("write your pallas kernel here")


# speedup vs baseline: 1.3170x; 1.3170x over previous
import functools
import math

import jax
import jax.numpy as jnp
from jax import lax
from jax.experimental import pallas as pl
from jax.experimental.pallas import tpu as pltpu
from jax.experimental.pallas import tpu_sc as plsc

D_MODEL = 32
SCALE = math.sqrt(float(D_MODEL))

NUM_CORES = 2
NUM_SUBCORES = 16
NUM_WORKERS = NUM_CORES * NUM_SUBCORES
LANES = 16

CHUNK = 2560


@functools.lru_cache(maxsize=None)
def _build(total_rows: int):
    assert total_rows % NUM_WORKERS == 0
    rows_per_worker = total_rows // NUM_WORKERS
    assert rows_per_worker % CHUNK == 0
    num_chunks = rows_per_worker // CHUNK

    mesh = plsc.VectorSubcoreMesh(core_axis_name="c", subcore_axis_name="s")

    @functools.partial(
        pl.kernel,
        mesh=mesh,
        out_type=jax.ShapeDtypeStruct((total_rows, D_MODEL), jnp.float32),
        scratch_types=[
            pltpu.VMEM((CHUNK,), jnp.int32),
            pltpu.VMEM((CHUNK, D_MODEL), jnp.float32),
            pltpu.SemaphoreType.DMA,
        ],
        compiler_params=pltpu.CompilerParams(use_tc_tiling_on_sc=False),
    )
    def gather_kernel(idx_hbm, table_hbm, out_hbm, idx_v, rows_v, sem):
        wid = lax.axis_index("s") * NUM_CORES + lax.axis_index("c")
        base = wid * rows_per_worker

        def chunk_body(c, carry):
            off = base + c * CHUNK
            pltpu.sync_copy(idx_hbm.at[pl.ds(off, CHUNK)], idx_v)
            pltpu.async_copy(table_hbm.at[idx_v], rows_v, sem).wait()

            def scale_row(r, carry2):
                rows_v[r, 0:LANES] = rows_v[r, 0:LANES] * SCALE
                rows_v[r, LANES:D_MODEL] = rows_v[r, LANES:D_MODEL] * SCALE
                return carry2

            lax.fori_loop(0, CHUNK, scale_row, 0)
            pltpu.sync_copy(rows_v, out_hbm.at[pl.ds(off, CHUNK)])
            return carry

        lax.fori_loop(0, num_chunks, chunk_body, 0)

    return gather_kernel


def kernel(inputs, embedding_matrix):
    b, s = inputs.shape
    idx = inputs.reshape(b * s).astype(jnp.int32)
    out = _build(b * s)(idx, embedding_matrix)
    return out.reshape(b, s, D_MODEL)


# speedup vs baseline: 1.4765x; 1.1211x over previous
import functools
import math

import jax
import jax.numpy as jnp
from jax import lax
from jax.experimental import pallas as pl
from jax.experimental.pallas import tpu as pltpu
from jax.experimental.pallas import tpu_sc as plsc

D_MODEL = 32
SCALE = math.sqrt(float(D_MODEL))

NUM_CORES = 2
NUM_SUBCORES = 16
NUM_WORKERS = NUM_CORES * NUM_SUBCORES
LANES = 16

CHUNK = 800
NRING = 4
LOOKAHEAD = 2
UNROLL = 8


@functools.lru_cache(maxsize=None)
def _build(total_rows: int):
    assert total_rows % NUM_WORKERS == 0
    rows_per_worker = total_rows // NUM_WORKERS
    assert rows_per_worker % CHUNK == 0
    num_chunks = rows_per_worker // CHUNK

    mesh = plsc.VectorSubcoreMesh(core_axis_name="c", subcore_axis_name="s")

    @functools.partial(
        pl.kernel,
        mesh=mesh,
        out_type=jax.ShapeDtypeStruct((total_rows, D_MODEL), jnp.float32),
        scratch_types=[pltpu.VMEM((rows_per_worker,), jnp.int32)]
        + [pltpu.VMEM((CHUNK, D_MODEL), jnp.float32) for _ in range(NRING)]
        + [pltpu.SemaphoreType.DMA for _ in range(2 * NRING)],
        compiler_params=pltpu.CompilerParams(use_tc_tiling_on_sc=False),
    )
    def gather_kernel(idx_hbm, table_hbm, out_hbm, idx_all, *scratch):
        rows = scratch[:NRING]
        gsem = scratch[NRING:2 * NRING]
        ssem = scratch[2 * NRING:]
        wid = lax.axis_index("s") * NUM_CORES + lax.axis_index("c")
        base = wid * rows_per_worker

        pltpu.sync_copy(idx_hbm.at[pl.ds(base, rows_per_worker)], idx_all)

        def issue_gather(c):
            b = c % NRING
            pltpu.async_copy(
                table_hbm.at[idx_all.at[pl.ds(c * CHUNK, CHUNK)]],
                rows[b], gsem[b])

        for c in range(min(LOOKAHEAD, num_chunks)):
            issue_gather(c)

        for c in range(num_chunks):
            b = c % NRING
            pltpu.make_async_copy(
                table_hbm.at[idx_all.at[pl.ds(c * CHUNK, CHUNK)]],
                rows[b], gsem[b]).wait()

            def scale_step(i, carry, b=b):
                r0 = i * UNROLL
                for u in range(UNROLL):
                    rows[b][r0 + u, 0:LANES] = rows[b][r0 + u, 0:LANES] * SCALE
                    rows[b][r0 + u, LANES:D_MODEL] = (
                        rows[b][r0 + u, LANES:D_MODEL] * SCALE)
                return carry

            lax.fori_loop(0, CHUNK // UNROLL, scale_step, 0)

            pltpu.async_copy(
                rows[b], out_hbm.at[pl.ds(base + c * CHUNK, CHUNK)], ssem[b])

            nxt = c + LOOKAHEAD
            if nxt < num_chunks:
                if c >= LOOKAHEAD:
                    pltpu.make_async_copy(
                        rows[nxt % NRING],
                        out_hbm.at[pl.ds(base + (nxt - NRING) * CHUNK, CHUNK)],
                        ssem[nxt % NRING]).wait()
                issue_gather(nxt)

        for c in range(max(0, num_chunks - NRING), num_chunks):
            b = c % NRING
            pltpu.make_async_copy(
                rows[b], out_hbm.at[pl.ds(base + c * CHUNK, CHUNK)],
                ssem[b]).wait()

    return gather_kernel


def kernel(inputs, embedding_matrix):
    b, s = inputs.shape
    idx = inputs.reshape(b * s).astype(jnp.int32)
    out = _build(b * s)(idx, embedding_matrix)
    return out.reshape(b, s, D_MODEL)
